# Initial kernel scaffold; baseline (speedup 1.0000x reference)
import jax
import jax.numpy as jnp
from jax.experimental import pallas as pl


def kernel(x, edge_index, W1, a_src1, a_dst1, W2, a_src2, a_dst2):
    raise NotImplementedError("write your pallas kernel here")


# speedup vs baseline: 22.0579x; 22.0579x over previous
import functools

import jax
import jax.numpy as jnp
from jax import lax
from jax.experimental import pallas as pl
from jax.experimental.pallas import tpu as pltpu
from jax.experimental.pallas import tpu_sc as plsc

N = 10000
IN_DIM = 128
HID = 128
HEADS = 4
OUT_DIM = 128

NC = 2
NS = 16
K = 128

NPAD = 10112
ZR = NPAD // NS
EP = 331776
CH1 = EP // NS
CH2 = EP // (NS * NC)

R_BLK = 400
GRID = N // R_BLK

_f32 = jnp.float32


def _t1_body(x_ref, w1_ref, asrc_ref, adst_ref, h2_ref, s1_ref, d1_ref, mx_ref):
    xb = x_ref[...]
    h2 = lax.dot_general(xb, w1_ref[...], (((1,), (1,)), ((), ())),
                         preferred_element_type=_f32)
    h2r = h2.reshape(R_BLK, HEADS, HID)
    s1 = jnp.sum(h2r * asrc_ref[...][None], axis=-1)
    d1 = jnp.sum(h2r * adst_ref[...][None], axis=-1)
    h2_ref[...] = jnp.transpose(h2r, (1, 0, 2))
    s1_ref[...] = s1
    d1_ref[...] = d1
    bm = jnp.concatenate([jnp.max(s1, axis=0), jnp.max(d1, axis=0)]).reshape(1, 8)

    @pl.when(pl.program_id(0) == 0)
    def _():
        mx_ref[...] = bm

    @pl.when(pl.program_id(0) > 0)
    def _():
        mx_ref[...] = jnp.maximum(mx_ref[...], bm)


def _t1_call(x, w1, asrc, adst):
    return pl.pallas_call(
        _t1_body,
        grid=(GRID,),
        in_specs=[
            pl.BlockSpec((R_BLK, IN_DIM), lambda i: (i, 0)),
            pl.BlockSpec((HEADS * HID, IN_DIM), lambda i: (0, 0)),
            pl.BlockSpec((HEADS, HID), lambda i: (0, 0)),
            pl.BlockSpec((HEADS, HID), lambda i: (0, 0)),
        ],
        out_specs=[
            pl.BlockSpec((HEADS, R_BLK, HID), lambda i: (0, i, 0)),
            pl.BlockSpec((R_BLK, HEADS), lambda i: (i, 0)),
            pl.BlockSpec((R_BLK, HEADS), lambda i: (i, 0)),
            pl.BlockSpec((1, 8), lambda i: (0, 0)),
        ],
        out_shape=[
            jax.ShapeDtypeStruct((HEADS, N, HID), _f32),
            jax.ShapeDtypeStruct((N, HEADS), _f32),
            jax.ShapeDtypeStruct((N, HEADS), _f32),
            jax.ShapeDtypeStruct((1, 8), _f32),
        ],
    )(x, w1, asrc, adst)


def _t2_body(acc_ref, s_ref, w2_ref, a2s_ref, a2d_ref, g_ref, sd_ref, mx_ref):
    a = acc_ref[...]
    sv = s_ref[...][:, :, 0]
    h1 = a / (sv[:, :, None] + 1e-16)
    h1 = jnp.maximum(h1, 0.0)
    h1 = jnp.transpose(h1, (1, 0, 2)).reshape(R_BLK, HEADS * HID)
    g = lax.dot_general(h1, w2_ref[...], (((1,), (1,)), ((), ())),
                        preferred_element_type=_f32)
    g_ref[...] = g
    s2 = jnp.sum(g * a2s_ref[...], axis=-1)
    d2 = jnp.sum(g * a2d_ref[...], axis=-1)
    sd = jnp.concatenate(
        [s2[:, None], d2[:, None], jnp.zeros((R_BLK, 6), _f32)], axis=1)
    sd_ref[...] = sd
    bm = jnp.concatenate(
        [jnp.max(s2)[None], jnp.max(d2)[None], jnp.zeros((6,), _f32)]).reshape(1, 8)

    @pl.when(pl.program_id(0) == 0)
    def _():
        mx_ref[...] = bm

    @pl.when(pl.program_id(0) > 0)
    def _():
        mx_ref[...] = jnp.maximum(mx_ref[...], bm)


def _t2_call(acc, s, w2, a2s, a2d):
    return pl.pallas_call(
        _t2_body,
        grid=(GRID,),
        in_specs=[
            pl.BlockSpec((HEADS, R_BLK, HID), lambda i: (0, i, 0)),
            pl.BlockSpec((HEADS, R_BLK, 16), lambda i: (0, i, 0)),
            pl.BlockSpec((OUT_DIM, HEADS * HID), lambda i: (0, 0)),
            pl.BlockSpec((1, OUT_DIM), lambda i: (0, 0)),
            pl.BlockSpec((1, OUT_DIM), lambda i: (0, 0)),
        ],
        out_specs=[
            pl.BlockSpec((R_BLK, OUT_DIM), lambda i: (i, 0)),
            pl.BlockSpec((R_BLK, 8), lambda i: (i, 0)),
            pl.BlockSpec((1, 8), lambda i: (0, 0)),
        ],
        out_shape=[
            jax.ShapeDtypeStruct((N, OUT_DIM), _f32),
            jax.ShapeDtypeStruct((N, 8), _f32),
            jax.ShapeDtypeStruct((1, 8), _f32),
        ],
    )(acc, s, w2, a2s, a2d)


def _t3_body(p_ref, s_ref, out_ref):
    p = p_ref[...]
    sv = s_ref[...][:, :, 0]
    out_ref[...] = (p[0] + p[1]) / (sv[0] + sv[1] + 1e-16)[:, None]


def _t3_call(p, s):
    return pl.pallas_call(
        _t3_body,
        grid=(GRID,),
        in_specs=[
            pl.BlockSpec((2, R_BLK, OUT_DIM), lambda i: (0, i, 0)),
            pl.BlockSpec((2, R_BLK, 16), lambda i: (0, i, 0)),
        ],
        out_specs=pl.BlockSpec((R_BLK, OUT_DIM), lambda i: (i, 0)),
        out_shape=jax.ShapeDtypeStruct((N, OUT_DIM), _f32),
    )(p, s)


def _sc1_body(h2cat, s1t, d1t, src_a, dst_a, shifts, zrow, zs,
              out, out_s,
              idx_src, idx_dst, idx_g, pbuf, rows, prows,
              s1buf, d1buf, shiftb, acc, sacc, sem):
    c = lax.axis_index("c")
    s = lax.axis_index("s")
    ebase = s * CH1
    zbase = s * ZR
    pltpu.sync_copy(shifts, shiftb)
    lanes0 = lax.iota(jnp.int32, 16) * 0

    for hh in range(2):
        h = c * 2 + hh
        pltpu.sync_copy(s1t.at[pl.ds(h * NPAD, NPAD)], s1buf)
        pltpu.sync_copy(d1t.at[pl.ds(h * NPAD, NPAD)], d1buf)
        pltpu.sync_copy(zrow, acc.at[pl.ds(zbase, ZR)])
        pltpu.sync_copy(zs, sacc.at[pl.ds(zbase, ZR)])
        plsc.subcore_barrier()
        sh_all = shiftb[...]
        shift_v = jnp.where(c == 0, sh_all[hh], sh_all[2 + hh])

        def window(w, carry):
            eb = ebase + w * K
            pltpu.sync_copy(src_a.at[pl.ds(eb, K)], idx_src)
            pltpu.sync_copy(dst_a.at[pl.ds(eb, K)], idx_dst)
            for i in range(K // 16):
                sl = pl.ds(i * 16, 16)
                sv = idx_src[sl]
                dv = idx_dst[sl]
                t = plsc.load_gather(s1buf, [sv]) + plsc.load_gather(d1buf, [dv])
                t = jnp.maximum(t, 0.2 * t) - shift_v
                pbuf[sl] = jnp.exp(t)
                idx_g[sl] = sv + h * N
            pltpu.async_copy(h2cat.at[idx_g], rows, sem).wait()

            def scale(k, carry2):
                pk = plsc.load_gather(pbuf, [lanes0 + k])
                for j in range(HID // 16):
                    slj = pl.ds(j * 16, 16)
                    rows[k, slj] = rows[k, slj] * pk
                prows[k, :] = pk
                return carry2

            lax.fori_loop(0, K, scale, 0)
            pltpu.sync_copy(rows, acc.at[idx_dst], add=True)
            pltpu.sync_copy(prows, sacc.at[idx_dst], add=True)
            return carry

        lax.fori_loop(0, CH1 // K, window, 0)
        plsc.subcore_barrier()
        pltpu.sync_copy(acc.at[pl.ds(zbase, ZR)],
                        out.at[pl.ds(h * NPAD + zbase, ZR)])
        pltpu.sync_copy(sacc.at[pl.ds(zbase, ZR)],
                        out_s.at[pl.ds(h * NPAD + zbase, ZR)])
        plsc.subcore_barrier()


def _sc2_body(g_hbm, s2t, d2t, src_a, dst_a, shifts, zrow, zs,
              out, out_s,
              idx_src, idx_dst, pbuf, rows, prows,
              s2buf, d2buf, shiftb, acc, sacc, sem):
    c = lax.axis_index("c")
    s = lax.axis_index("s")
    wid = c * NS + s
    ebase = wid * CH2
    zbase = s * ZR
    pltpu.sync_copy(shifts, shiftb)
    lanes0 = lax.iota(jnp.int32, 16) * 0
    pltpu.sync_copy(s2t, s2buf)
    pltpu.sync_copy(d2t, d2buf)
    pltpu.sync_copy(zrow, acc.at[pl.ds(zbase, ZR)])
    pltpu.sync_copy(zs, sacc.at[pl.ds(zbase, ZR)])
    plsc.subcore_barrier()
    shift_v = shiftb[...][0]

    def window(w, carry):
        eb = ebase + w * K
        pltpu.sync_copy(src_a.at[pl.ds(eb, K)], idx_src)
        pltpu.sync_copy(dst_a.at[pl.ds(eb, K)], idx_dst)
        for i in range(K // 16):
            sl = pl.ds(i * 16, 16)
            sv = idx_src[sl]
            dv = idx_dst[sl]
            t = plsc.load_gather(s2buf, [sv]) + plsc.load_gather(d2buf, [dv])
            t = jnp.maximum(t, 0.2 * t) - shift_v
            pbuf[sl] = jnp.exp(t)
        pltpu.async_copy(g_hbm.at[idx_src], rows, sem).wait()

        def scale(k, carry2):
            pk = plsc.load_gather(pbuf, [lanes0 + k])
            for j in range(OUT_DIM // 16):
                slj = pl.ds(j * 16, 16)
                rows[k, slj] = rows[k, slj] * pk
            prows[k, :] = pk
            return carry2

        lax.fori_loop(0, K, scale, 0)
        pltpu.sync_copy(rows, acc.at[idx_dst], add=True)
        pltpu.sync_copy(prows, sacc.at[idx_dst], add=True)
        return carry

    lax.fori_loop(0, CH2 // K, window, 0)
    plsc.subcore_barrier()
    pltpu.sync_copy(acc.at[pl.ds(zbase, ZR)],
                    out.at[pl.ds(c * NPAD + zbase, ZR)])
    pltpu.sync_copy(sacc.at[pl.ds(zbase, ZR)],
                    out_s.at[pl.ds(c * NPAD + zbase, ZR)])


_MESH = plsc.VectorSubcoreMesh(
    core_axis_name="c", subcore_axis_name="s", num_cores=NC, num_subcores=NS)

_SC_PARAMS = pltpu.CompilerParams(
    needs_layout_passes=False, use_tc_tiling_on_sc=False)


def _sc1_call(h2flat, s1t, d1t, srcp, dstp, shifts, zrow, zs):
    return pl.kernel(
        _sc1_body,
        out_type=[
            jax.ShapeDtypeStruct((HEADS * NPAD, HID), _f32),
            jax.ShapeDtypeStruct((HEADS * NPAD, 16), _f32),
        ],
        mesh=_MESH,
        scratch_types=[
            pltpu.VMEM((K,), jnp.int32),
            pltpu.VMEM((K,), jnp.int32),
            pltpu.VMEM((K,), jnp.int32),
            pltpu.VMEM((K,), _f32),
            pltpu.VMEM((K, HID), _f32),
            pltpu.VMEM((K, 16), _f32),
            pltpu.VMEM((NPAD,), _f32),
            pltpu.VMEM((NPAD,), _f32),
            pltpu.VMEM((16,), _f32),
            pltpu.VMEM_SHARED((NPAD, HID), _f32),
            pltpu.VMEM_SHARED((NPAD, 16), _f32),
            pltpu.SemaphoreType.DMA,
        ],
        compiler_params=_SC_PARAMS,
    )(h2flat, s1t, d1t, srcp, dstp, shifts, zrow, zs)


def _sc2_call(g, s2t, d2t, srcp, dstp, shifts, zrow, zs):
    return pl.kernel(
        _sc2_body,
        out_type=[
            jax.ShapeDtypeStruct((NC * NPAD, OUT_DIM), _f32),
            jax.ShapeDtypeStruct((NC * NPAD, 16), _f32),
        ],
        mesh=_MESH,
        scratch_types=[
            pltpu.VMEM((K,), jnp.int32),
            pltpu.VMEM((K,), jnp.int32),
            pltpu.VMEM((K,), _f32),
            pltpu.VMEM((K, OUT_DIM), _f32),
            pltpu.VMEM((K, 16), _f32),
            pltpu.VMEM((NPAD,), _f32),
            pltpu.VMEM((NPAD,), _f32),
            pltpu.VMEM((16,), _f32),
            pltpu.VMEM_SHARED((NPAD, OUT_DIM), _f32),
            pltpu.VMEM_SHARED((NPAD, 16), _f32),
            pltpu.SemaphoreType.DMA,
        ],
        compiler_params=_SC_PARAMS,
    )(g, s2t, d2t, srcp, dstp, shifts, zrow, zs)


@jax.jit
def kernel(x, edge_index, W1, a_src1, a_dst1, W2, a_src2, a_dst2):
    loop = jnp.arange(N, dtype=jnp.int32)
    ei = edge_index.astype(jnp.int32)
    src = jnp.concatenate([ei[0], loop])
    dst = jnp.concatenate([ei[1], loop])
    pad = EP - src.shape[0]
    srcp = jnp.concatenate([src, jnp.zeros((pad,), jnp.int32)])
    dstp = jnp.concatenate([dst, jnp.full((pad,), N, jnp.int32)])
    zrow = jnp.zeros((ZR, HID), _f32)
    zs = jnp.zeros((ZR, 16), _f32)

    h2cat, s1, d1, mx1 = _t1_call(x, W1, a_src1, a_dst1)
    h2flat = h2cat.reshape(HEADS * N, HID)
    s1t = jnp.concatenate([s1.T, jnp.zeros((HEADS, NPAD - N), _f32)], 1).reshape(-1)
    d1t = jnp.concatenate([d1.T, jnp.full((HEADS, NPAD - N), -1e30, _f32)], 1).reshape(-1)
    m1 = mx1[0, :4] + mx1[0, 4:]
    shift1 = jnp.maximum(m1, 0.2 * m1)
    shifts1 = jnp.concatenate([shift1, jnp.zeros((12,), _f32)])

    acc1, s1acc = _sc1_call(h2flat, s1t, d1t, srcp, dstp, shifts1, zrow, zs)

    g, sd, mx2 = _t2_call(acc1.reshape(HEADS, NPAD, HID),
                          s1acc.reshape(HEADS, NPAD, 16), W2, a_src2, a_dst2)
    s2t = jnp.concatenate([sd[:, 0], jnp.zeros((NPAD - N,), _f32)])
    d2t = jnp.concatenate([sd[:, 1], jnp.full((NPAD - N,), -1e30, _f32)])
    m2 = mx2[0, 0] + mx2[0, 1]
    shift2 = jnp.maximum(m2, 0.2 * m2)
    shifts2 = jnp.full((16,), shift2, _f32)

    acc2, s2acc = _sc2_call(g, s2t, d2t, srcp, dstp, shifts2, zrow, zs)

    return _t3_call(acc2.reshape(NC, NPAD, OUT_DIM),
                    s2acc.reshape(NC, NPAD, 16))


# speedup vs baseline: 29.7562x; 1.3490x over previous
import jax
import jax.numpy as jnp
from jax import lax
from jax.experimental import pallas as pl
from jax.experimental.pallas import tpu as pltpu
from jax.experimental.pallas import tpu_sc as plsc

N = 10000
IN_DIM = 128
HID = 128
HEADS = 4
OUT_DIM = 128

NC = 2
NS = 16
K = 128

NPAD = 10112
ZR = NPAD // NS
EP = 335872
W1C = EP // (NS * K)
W2C = EP // (NS * NC * K)

R_BLK = 400
GRID = N // R_BLK

_f32 = jnp.float32


def _t1_body(x_ref, w1_ref, asrc_ref, adst_ref, h2_ref, s1_ref, d1_ref, mx_ref):
    xb = x_ref[...]
    h2 = lax.dot_general(xb, w1_ref[...], (((1,), (1,)), ((), ())),
                         preferred_element_type=_f32)
    h2r = h2.reshape(R_BLK, HEADS, HID)
    s1 = jnp.sum(h2r * asrc_ref[...][None], axis=-1)
    d1 = jnp.sum(h2r * adst_ref[...][None], axis=-1)
    h2_ref[...] = jnp.transpose(h2r, (1, 0, 2))
    s1_ref[...] = s1
    d1_ref[...] = d1
    bm = jnp.concatenate([jnp.max(s1, axis=0), jnp.max(d1, axis=0)]).reshape(1, 8)

    @pl.when(pl.program_id(0) == 0)
    def _():
        mx_ref[...] = bm

    @pl.when(pl.program_id(0) > 0)
    def _():
        mx_ref[...] = jnp.maximum(mx_ref[...], bm)


def _t1_call(x, w1, asrc, adst):
    return pl.pallas_call(
        _t1_body,
        grid=(GRID,),
        in_specs=[
            pl.BlockSpec((R_BLK, IN_DIM), lambda i: (i, 0)),
            pl.BlockSpec((HEADS * HID, IN_DIM), lambda i: (0, 0)),
            pl.BlockSpec((HEADS, HID), lambda i: (0, 0)),
            pl.BlockSpec((HEADS, HID), lambda i: (0, 0)),
        ],
        out_specs=[
            pl.BlockSpec((HEADS, R_BLK, HID), lambda i: (0, i, 0)),
            pl.BlockSpec((R_BLK, HEADS), lambda i: (i, 0)),
            pl.BlockSpec((R_BLK, HEADS), lambda i: (i, 0)),
            pl.BlockSpec((1, 8), lambda i: (0, 0)),
        ],
        out_shape=[
            jax.ShapeDtypeStruct((HEADS, N, HID), _f32),
            jax.ShapeDtypeStruct((N, HEADS), _f32),
            jax.ShapeDtypeStruct((N, HEADS), _f32),
            jax.ShapeDtypeStruct((1, 8), _f32),
        ],
    )(x, w1, asrc, adst)


def _t2_body(acc_ref, s_ref, w2_ref, a2s_ref, a2d_ref, g_ref, sd_ref, mx_ref):
    a = acc_ref[...]
    i = pl.program_id(0)
    sv = s_ref[:, pl.ds(i * (R_BLK // 16), R_BLK // 16), :]
    a4 = a.reshape(HEADS, R_BLK // 16, 16, HID)
    h1 = a4 / (sv[:, :, :, None] + 1e-16)
    h1 = jnp.maximum(h1, 0.0).reshape(HEADS, R_BLK, HID)
    h1 = jnp.transpose(h1, (1, 0, 2)).reshape(R_BLK, HEADS * HID)
    g = lax.dot_general(h1, w2_ref[...], (((1,), (1,)), ((), ())),
                        preferred_element_type=_f32)
    g_ref[...] = g
    s2 = jnp.sum(g * a2s_ref[...], axis=-1)
    d2 = jnp.sum(g * a2d_ref[...], axis=-1)
    sd = jnp.concatenate(
        [s2[:, None], d2[:, None], jnp.zeros((R_BLK, 6), _f32)], axis=1)
    sd_ref[...] = sd
    bm = jnp.concatenate(
        [jnp.max(s2)[None], jnp.max(d2)[None], jnp.zeros((6,), _f32)]).reshape(1, 8)

    @pl.when(pl.program_id(0) == 0)
    def _():
        mx_ref[...] = bm

    @pl.when(pl.program_id(0) > 0)
    def _():
        mx_ref[...] = jnp.maximum(mx_ref[...], bm)


def _t2_call(acc, s, w2, a2s, a2d):
    return pl.pallas_call(
        _t2_body,
        grid=(GRID,),
        in_specs=[
            pl.BlockSpec((HEADS, R_BLK, HID), lambda i: (0, i, 0)),
            pl.BlockSpec((HEADS, NPAD // 16, 16), lambda i: (0, 0, 0)),
            pl.BlockSpec((OUT_DIM, HEADS * HID), lambda i: (0, 0)),
            pl.BlockSpec((1, OUT_DIM), lambda i: (0, 0)),
            pl.BlockSpec((1, OUT_DIM), lambda i: (0, 0)),
        ],
        out_specs=[
            pl.BlockSpec((R_BLK, OUT_DIM), lambda i: (i, 0)),
            pl.BlockSpec((R_BLK, 8), lambda i: (i, 0)),
            pl.BlockSpec((1, 8), lambda i: (0, 0)),
        ],
        out_shape=[
            jax.ShapeDtypeStruct((N, OUT_DIM), _f32),
            jax.ShapeDtypeStruct((N, 8), _f32),
            jax.ShapeDtypeStruct((1, 8), _f32),
        ],
    )(acc, s, w2, a2s, a2d)


def _t3_body(p_ref, s_ref, out_ref):
    p = p_ref[...]
    i = pl.program_id(0)
    sv = s_ref[:, pl.ds(i * (R_BLK // 16), R_BLK // 16), :]
    p4 = p.reshape(NC, R_BLK // 16, 16, OUT_DIM)
    o = (p4[0] + p4[1]) / (sv[0] + sv[1] + 1e-16)[:, :, None]
    out_ref[...] = o.reshape(R_BLK, OUT_DIM)


def _t3_call(p, s):
    return pl.pallas_call(
        _t3_body,
        grid=(GRID,),
        in_specs=[
            pl.BlockSpec((NC, R_BLK, OUT_DIM), lambda i: (0, i, 0)),
            pl.BlockSpec((NC, NPAD // 16, 16), lambda i: (0, 0, 0)),
        ],
        out_specs=pl.BlockSpec((R_BLK, OUT_DIM), lambda i: (i, 0)),
        out_shape=jax.ShapeDtypeStruct((N, OUT_DIM), _f32),
    )(p, s)


def _edge_phase(h2cat, src_a, dst_a, rowbase, nwin, hoff, shift_v,
                s1sp, d1sp, bufs, acc, sacc, sems, lanes0):
    isrc, idst, idxg, sidx, tsrc, tdst, pbuf, rows = bufs
    sem_i, sem_t, sem_g, sem_r, sem_p = sems

    def issue_idx(w, b):
        pltpu.async_copy(src_a.at[pl.ds(rowbase + w, 1)], isrc.at[pl.ds(b, 1)],
                         sem_i[b])
        pltpu.async_copy(dst_a.at[pl.ds(rowbase + w, 1)], idst.at[pl.ds(b, 1)],
                         sem_i[b])

    def wait_idx(b):
        pltpu.make_async_copy(src_a.at[pl.ds(0, 1)], isrc.at[pl.ds(b, 1)],
                              sem_i[b]).wait()
        pltpu.make_async_copy(dst_a.at[pl.ds(0, 1)], idst.at[pl.ds(b, 1)],
                              sem_i[b]).wait()

    def issue_gathers(b):
        if hoff is not None:
            for i in range(K // 16):
                sl = pl.ds(i * 16, 16)
                idxg[b, sl] = isrc[b, sl] + hoff
            gidx = idxg.at[b]
        else:
            gidx = isrc.at[b]
        pltpu.async_copy(h2cat.at[gidx], rows.at[b], sem_g[b])
        pltpu.async_copy(s1sp.at[isrc.at[b]], tsrc.at[b], sem_t[b])
        pltpu.async_copy(d1sp.at[idst.at[b]], tdst.at[b], sem_t[b])

    def wait_gathers(b):
        pltpu.make_async_copy(h2cat.at[idxg.at[b]], rows.at[b], sem_g[b]).wait()
        pltpu.make_async_copy(s1sp.at[isrc.at[b]], tsrc.at[b], sem_t[b]).wait()
        pltpu.make_async_copy(d1sp.at[idst.at[b]], tdst.at[b], sem_t[b]).wait()

    def wait_scatters(b):
        pltpu.make_async_copy(rows.at[b], acc.at[sidx.at[b]], sem_r[b]).wait()
        pltpu.make_async_copy(pbuf.at[b], sacc.at[sidx.at[b]], sem_p[b]).wait()

    issue_idx(0, 0)
    issue_idx(1, 1)
    wait_idx(0)
    issue_gathers(0)

    def pair(wp, carry):
        for b in range(2):
            w = wp * 2 + b
            nb = 1 - b

            @pl.when(w + 1 < nwin)
            def _():
                wait_idx(nb)

                @pl.when(w >= 1)
                def _():
                    wait_scatters(nb)

                issue_gathers(nb)

            wait_gathers(b)
            for i in range(K // 16):
                sl = pl.ds(i * 16, 16)
                sidx[b, sl] = idst[b, sl]
                t = tsrc[b, sl] + tdst[b, sl]
                t = jnp.maximum(t, 0.2 * t) - shift_v
                pbuf[b, sl] = jnp.exp(t)

            def scale(k, carry2):
                pk = plsc.load_gather(pbuf.at[b], [lanes0 + k])
                for j in range(HID // 16):
                    slj = pl.ds(j * 16, 16)
                    rows[b, k, slj] = rows[b, k, slj] * pk
                return carry2

            lax.fori_loop(0, K, scale, 0)
            pltpu.async_copy(rows.at[b], acc.at[sidx.at[b]], sem_r[b],
                             add=True)
            pltpu.async_copy(pbuf.at[b], sacc.at[sidx.at[b]], sem_p[b],
                             add=True)

            @pl.when(w + 2 < nwin)
            def _():
                issue_idx(w + 2, b)
        return carry

    lax.fori_loop(0, nwin // 2, pair, 0)
    wait_scatters(0)
    wait_scatters(1)


def _sc1_body(h2cat, s1t, d1t, src_a, dst_a, shifts, zrow, zs,
              out, out_s,
              isrc, idst, idxg, sidx, tsrc, tdst, pbuf, rows, shiftb,
              s1sp, d1sp, acc, sacc,
              sem_i0, sem_i1, sem_t0, sem_t1, sem_g0, sem_g1,
              sem_r0, sem_r1, sem_p0, sem_p1):
    c = lax.axis_index("c")
    s = lax.axis_index("s")
    zbase = s * ZR
    lanes0 = lax.iota(jnp.int32, 16) * 0
    pltpu.sync_copy(shifts, shiftb)
    sh_all = shiftb[...]
    bufs = (isrc, idst, idxg, sidx, tsrc, tdst, pbuf, rows)
    sems = ((sem_i0, sem_i1), (sem_t0, sem_t1), (sem_g0, sem_g1),
            (sem_r0, sem_r1), (sem_p0, sem_p1))

    for hh in range(2):
        h = c * 2 + hh
        shift_v = jnp.where(c == 0, sh_all[hh], sh_all[2 + hh])
        pltpu.sync_copy(s1t.at[pl.ds(h * NPAD + zbase, ZR)],
                        s1sp.at[pl.ds(zbase, ZR)])
        pltpu.sync_copy(d1t.at[pl.ds(h * NPAD + zbase, ZR)],
                        d1sp.at[pl.ds(zbase, ZR)])
        pltpu.sync_copy(zrow, acc.at[pl.ds(zbase, ZR)])
        pltpu.sync_copy(zs, sacc.at[pl.ds(zbase, ZR)])
        plsc.subcore_barrier()

        _edge_phase(h2cat, src_a, dst_a, s * W1C, W1C, h * N, shift_v,
                    s1sp, d1sp, bufs, acc, sacc, sems, lanes0)

        plsc.subcore_barrier()
        pltpu.sync_copy(acc.at[pl.ds(zbase, ZR)],
                        out.at[pl.ds(h * NPAD + zbase, ZR)])
        pltpu.sync_copy(sacc.at[pl.ds(zbase, ZR)],
                        out_s.at[pl.ds(h * NPAD + zbase, ZR)])
        plsc.subcore_barrier()


def _sc2_body(g_hbm, s2t, d2t, src_a, dst_a, shifts, zrow, zs,
              out, out_s,
              isrc, idst, idxg, sidx, tsrc, tdst, pbuf, rows, shiftb,
              s2sp, d2sp, acc, sacc,
              sem_i0, sem_i1, sem_t0, sem_t1, sem_g0, sem_g1,
              sem_r0, sem_r1, sem_p0, sem_p1):
    c = lax.axis_index("c")
    s = lax.axis_index("s")
    wid = c * NS + s
    zbase = s * ZR
    lanes0 = lax.iota(jnp.int32, 16) * 0
    pltpu.sync_copy(shifts, shiftb)
    shift_v = shiftb[...][0]
    bufs = (isrc, idst, idxg, sidx, tsrc, tdst, pbuf, rows)
    sems = ((sem_i0, sem_i1), (sem_t0, sem_t1), (sem_g0, sem_g1),
            (sem_r0, sem_r1), (sem_p0, sem_p1))

    pltpu.sync_copy(s2t.at[pl.ds(zbase, ZR)], s2sp.at[pl.ds(zbase, ZR)])
    pltpu.sync_copy(d2t.at[pl.ds(zbase, ZR)], d2sp.at[pl.ds(zbase, ZR)])
    pltpu.sync_copy(zrow, acc.at[pl.ds(zbase, ZR)])
    pltpu.sync_copy(zs, sacc.at[pl.ds(zbase, ZR)])
    plsc.subcore_barrier()

    _edge_phase(g_hbm, src_a, dst_a, wid * W2C, W2C, None, shift_v,
                s2sp, d2sp, bufs, acc, sacc, sems, lanes0)

    plsc.subcore_barrier()
    pltpu.sync_copy(acc.at[pl.ds(zbase, ZR)],
                    out.at[pl.ds(c * NPAD + zbase, ZR)])
    pltpu.sync_copy(sacc.at[pl.ds(zbase, ZR)],
                    out_s.at[pl.ds(c * NPAD + zbase, ZR)])


_MESH = plsc.VectorSubcoreMesh(
    core_axis_name="c", subcore_axis_name="s", num_cores=NC, num_subcores=NS)

_SC_PARAMS = pltpu.CompilerParams(
    needs_layout_passes=False, use_tc_tiling_on_sc=False)

_SC_SEMS = [pltpu.SemaphoreType.DMA] * 10

_SC_SCRATCH = [
    pltpu.VMEM((2, K), jnp.int32),
    pltpu.VMEM((2, K), jnp.int32),
    pltpu.VMEM((2, K), jnp.int32),
    pltpu.VMEM((2, K), jnp.int32),
    pltpu.VMEM((2, K), _f32),
    pltpu.VMEM((2, K), _f32),
    pltpu.VMEM((2, K), _f32),
    pltpu.VMEM((2, K, HID), _f32),
    pltpu.VMEM((16,), _f32),
    pltpu.VMEM_SHARED((NPAD,), _f32),
    pltpu.VMEM_SHARED((NPAD,), _f32),
    pltpu.VMEM_SHARED((NPAD, HID), _f32),
    pltpu.VMEM_SHARED((NPAD,), _f32),
] + _SC_SEMS


def _sc1_call(h2flat, s1t, d1t, srcp, dstp, shifts, zrow, zs):
    return pl.kernel(
        _sc1_body,
        out_type=[
            jax.ShapeDtypeStruct((HEADS * NPAD, HID), _f32),
            jax.ShapeDtypeStruct((HEADS * NPAD,), _f32),
        ],
        mesh=_MESH,
        scratch_types=_SC_SCRATCH,
        compiler_params=_SC_PARAMS,
    )(h2flat, s1t, d1t, srcp, dstp, shifts, zrow, zs)


def _sc2_call(g, s2t, d2t, srcp, dstp, shifts, zrow, zs):
    return pl.kernel(
        _sc2_body,
        out_type=[
            jax.ShapeDtypeStruct((NC * NPAD, OUT_DIM), _f32),
            jax.ShapeDtypeStruct((NC * NPAD,), _f32),
        ],
        mesh=_MESH,
        scratch_types=_SC_SCRATCH,
        compiler_params=_SC_PARAMS,
    )(g, s2t, d2t, srcp, dstp, shifts, zrow, zs)


@jax.jit
def kernel(x, edge_index, W1, a_src1, a_dst1, W2, a_src2, a_dst2):
    loop = jnp.arange(N, dtype=jnp.int32)
    ei = edge_index.astype(jnp.int32)
    src = jnp.concatenate([ei[0], loop])
    dst = jnp.concatenate([ei[1], loop])
    pad = EP - src.shape[0]
    srcp = jnp.concatenate([src, jnp.zeros((pad,), jnp.int32)]).reshape(-1, K)
    dstp = jnp.concatenate([dst, jnp.full((pad,), N, jnp.int32)]).reshape(-1, K)
    zrow = jnp.zeros((ZR, HID), _f32)
    zs = jnp.zeros((ZR,), _f32)

    h2cat, s1, d1, mx1 = _t1_call(x, W1, a_src1, a_dst1)
    h2flat = h2cat.reshape(HEADS * N, HID)
    s1t = jnp.concatenate([s1.T, jnp.zeros((HEADS, NPAD - N), _f32)], 1).reshape(-1)
    d1t = jnp.concatenate([d1.T, jnp.full((HEADS, NPAD - N), -1e30, _f32)], 1).reshape(-1)
    m1 = mx1[0, :4] + mx1[0, 4:]
    shift1 = jnp.maximum(m1, 0.2 * m1)
    shifts1 = jnp.concatenate([shift1, jnp.zeros((12,), _f32)])

    acc1, s1acc = _sc1_call(h2flat, s1t, d1t, srcp, dstp, shifts1, zrow, zs)

    g, sd, mx2 = _t2_call(acc1.reshape(HEADS, NPAD, HID),
                          s1acc.reshape(HEADS, NPAD // 16, 16),
                          W2, a_src2, a_dst2)
    s2t = jnp.concatenate([sd[:, 0], jnp.zeros((NPAD - N,), _f32)])
    d2t = jnp.concatenate([sd[:, 1], jnp.full((NPAD - N,), -1e30, _f32)])
    m2 = mx2[0, 0] + mx2[0, 1]
    shift2 = jnp.maximum(m2, 0.2 * m2)
    shifts2 = jnp.full((16,), shift2, _f32)

    acc2, s2acc = _sc2_call(g, s2t, d2t, srcp, dstp, shifts2, zrow, zs)

    return _t3_call(acc2.reshape(NC, NPAD, OUT_DIM),
                    s2acc.reshape(NC, NPAD // 16, 16))


# speedup vs baseline: 31.9201x; 1.0727x over previous
import jax
import jax.numpy as jnp
from jax import lax
from jax.experimental import pallas as pl
from jax.experimental.pallas import tpu as pltpu
from jax.experimental.pallas import tpu_sc as plsc

N = 10000
IN_DIM = 128
HID = 128
HEADS = 4
OUT_DIM = 128

NC = 2
NS = 16
K = 128

NPAD = 10112
ZR = NPAD // NS
EP = 335872
W1C = EP // (NS * K)
W2C = EP // (NS * NC * K)

R_BLK = 400
GRID = N // R_BLK

_f32 = jnp.float32


def _t1_body(x_ref, w1_ref, asrc_ref, adst_ref, h2_ref, s1_ref, d1_ref, mx_ref):
    xb = x_ref[...]
    h2 = lax.dot_general(xb, w1_ref[...], (((1,), (1,)), ((), ())),
                         preferred_element_type=_f32)
    h2r = h2.reshape(R_BLK, HEADS, HID)
    s1 = jnp.sum(h2r * asrc_ref[...][None], axis=-1)
    d1 = jnp.sum(h2r * adst_ref[...][None], axis=-1)
    h2_ref[...] = jnp.transpose(h2r, (1, 0, 2))
    s1_ref[...] = s1
    d1_ref[...] = d1
    bm = jnp.concatenate([jnp.max(s1, axis=0), jnp.max(d1, axis=0)]).reshape(1, 8)

    @pl.when(pl.program_id(0) == 0)
    def _():
        mx_ref[...] = bm

    @pl.when(pl.program_id(0) > 0)
    def _():
        mx_ref[...] = jnp.maximum(mx_ref[...], bm)


def _t1_call(x, w1, asrc, adst):
    return pl.pallas_call(
        _t1_body,
        grid=(GRID,),
        in_specs=[
            pl.BlockSpec((R_BLK, IN_DIM), lambda i: (i, 0)),
            pl.BlockSpec((HEADS * HID, IN_DIM), lambda i: (0, 0)),
            pl.BlockSpec((HEADS, HID), lambda i: (0, 0)),
            pl.BlockSpec((HEADS, HID), lambda i: (0, 0)),
        ],
        out_specs=[
            pl.BlockSpec((HEADS, R_BLK, HID), lambda i: (0, i, 0)),
            pl.BlockSpec((R_BLK, HEADS), lambda i: (i, 0)),
            pl.BlockSpec((R_BLK, HEADS), lambda i: (i, 0)),
            pl.BlockSpec((1, 8), lambda i: (0, 0)),
        ],
        out_shape=[
            jax.ShapeDtypeStruct((HEADS, N, HID), _f32),
            jax.ShapeDtypeStruct((N, HEADS), _f32),
            jax.ShapeDtypeStruct((N, HEADS), _f32),
            jax.ShapeDtypeStruct((1, 8), _f32),
        ],
    )(x, w1, asrc, adst)


def _t2_body(acc_ref, s_ref, w2_ref, a2s_ref, a2d_ref, g_ref, sd_ref, mx_ref):
    a = acc_ref[...]
    i = pl.program_id(0)
    sv = s_ref[:, pl.ds(i * (R_BLK // 16), R_BLK // 16), :]
    a4 = a.reshape(HEADS, R_BLK // 16, 16, HID)
    h1 = a4 / (sv[:, :, :, None] + 1e-16)
    h1 = jnp.maximum(h1, 0.0).reshape(HEADS, R_BLK, HID)
    h1 = jnp.transpose(h1, (1, 0, 2)).reshape(R_BLK, HEADS * HID)
    g = lax.dot_general(h1, w2_ref[...], (((1,), (1,)), ((), ())),
                        preferred_element_type=_f32)
    g_ref[...] = g
    s2 = jnp.sum(g * a2s_ref[...], axis=-1)
    d2 = jnp.sum(g * a2d_ref[...], axis=-1)
    sd = jnp.concatenate(
        [s2[:, None], d2[:, None], jnp.zeros((R_BLK, 6), _f32)], axis=1)
    sd_ref[...] = sd
    bm = jnp.concatenate(
        [jnp.max(s2)[None], jnp.max(d2)[None], jnp.zeros((6,), _f32)]).reshape(1, 8)

    @pl.when(pl.program_id(0) == 0)
    def _():
        mx_ref[...] = bm

    @pl.when(pl.program_id(0) > 0)
    def _():
        mx_ref[...] = jnp.maximum(mx_ref[...], bm)


def _t2_call(acc, s, w2, a2s, a2d):
    return pl.pallas_call(
        _t2_body,
        grid=(GRID,),
        in_specs=[
            pl.BlockSpec((HEADS, R_BLK, HID), lambda i: (0, i, 0)),
            pl.BlockSpec((HEADS, NPAD // 16, 16), lambda i: (0, 0, 0)),
            pl.BlockSpec((OUT_DIM, HEADS * HID), lambda i: (0, 0)),
            pl.BlockSpec((1, OUT_DIM), lambda i: (0, 0)),
            pl.BlockSpec((1, OUT_DIM), lambda i: (0, 0)),
        ],
        out_specs=[
            pl.BlockSpec((R_BLK, OUT_DIM), lambda i: (i, 0)),
            pl.BlockSpec((R_BLK, 8), lambda i: (i, 0)),
            pl.BlockSpec((1, 8), lambda i: (0, 0)),
        ],
        out_shape=[
            jax.ShapeDtypeStruct((N, OUT_DIM), _f32),
            jax.ShapeDtypeStruct((N, 8), _f32),
            jax.ShapeDtypeStruct((1, 8), _f32),
        ],
    )(acc, s, w2, a2s, a2d)


def _t3_body(p_ref, s_ref, out_ref):
    p = p_ref[...]
    i = pl.program_id(0)
    sv = s_ref[:, pl.ds(i * (R_BLK // 16), R_BLK // 16), :]
    p4 = p.reshape(NC, R_BLK // 16, 16, OUT_DIM)
    o = (p4[0] + p4[1]) / (sv[0] + sv[1] + 1e-16)[:, :, None]
    out_ref[...] = o.reshape(R_BLK, OUT_DIM)


def _t3_call(p, s):
    return pl.pallas_call(
        _t3_body,
        grid=(GRID,),
        in_specs=[
            pl.BlockSpec((NC, R_BLK, OUT_DIM), lambda i: (0, i, 0)),
            pl.BlockSpec((NC, NPAD // 16, 16), lambda i: (0, 0, 0)),
        ],
        out_specs=pl.BlockSpec((R_BLK, OUT_DIM), lambda i: (i, 0)),
        out_shape=jax.ShapeDtypeStruct((N, OUT_DIM), _f32),
    )(p, s)


def _edge_phase(h2cat, src_a, dst_a, rowbase, nwin, hoff, shift_v,
                s1sp, d1sp, bufs, acc, sacc, sems, lanes0):
    isrc, idst, idxg, sidx, tsrc, tdst, pbuf, rows = bufs
    sem_i, sem_t, sem_g, sem_r, sem_p = sems

    def issue_idx(w, b):
        pltpu.async_copy(src_a.at[pl.ds(rowbase + w, 1)], isrc.at[pl.ds(b, 1)],
                         sem_i[b])
        pltpu.async_copy(dst_a.at[pl.ds(rowbase + w, 1)], idst.at[pl.ds(b, 1)],
                         sem_i[b])

    def wait_idx(b):
        pltpu.make_async_copy(src_a.at[pl.ds(0, 1)], isrc.at[pl.ds(b, 1)],
                              sem_i[b]).wait()
        pltpu.make_async_copy(dst_a.at[pl.ds(0, 1)], idst.at[pl.ds(b, 1)],
                              sem_i[b]).wait()

    def issue_gathers(b):
        if hoff is not None:
            for i in range(K // 16):
                sl = pl.ds(i * 16, 16)
                idxg[b, sl] = isrc[b, sl] + hoff
            gidx = idxg.at[b]
        else:
            gidx = isrc.at[b]
        pltpu.async_copy(h2cat.at[gidx], rows.at[b], sem_g[b])
        pltpu.async_copy(s1sp.at[isrc.at[b]], tsrc.at[b], sem_t[b])
        pltpu.async_copy(d1sp.at[idst.at[b]], tdst.at[b], sem_t[b])

    def wait_gathers(b):
        pltpu.make_async_copy(h2cat.at[idxg.at[b]], rows.at[b], sem_g[b]).wait()
        pltpu.make_async_copy(s1sp.at[isrc.at[b]], tsrc.at[b], sem_t[b]).wait()
        pltpu.make_async_copy(d1sp.at[idst.at[b]], tdst.at[b], sem_t[b]).wait()

    def wait_scatters(b):
        pltpu.make_async_copy(rows.at[b], acc.at[sidx.at[b]], sem_r[b]).wait()
        pltpu.make_async_copy(pbuf.at[b], sacc.at[sidx.at[b]], sem_p[b]).wait()

    issue_idx(0, 0)
    issue_idx(1, 1)
    wait_idx(0)
    issue_gathers(0)

    def pair(wp, carry):
        for b in range(2):
            w = wp * 2 + b
            nb = 1 - b

            @pl.when(w + 1 < nwin)
            def _():
                wait_idx(nb)

                @pl.when(w >= 1)
                def _():
                    wait_scatters(nb)

                issue_gathers(nb)

            wait_gathers(b)
            for i in range(K // 16):
                sl = pl.ds(i * 16, 16)
                sidx[b, sl] = idst[b, sl]
                t = tsrc[b, sl] + tdst[b, sl]
                t = jnp.maximum(t, 0.2 * t) - shift_v
                pbuf[b, sl] = jnp.exp(t)

            @plsc.parallel_loop(0, K, unroll=8)
            def _(k):
                pk = plsc.load_gather(pbuf.at[b], [lanes0 + k])
                for j in range(HID // 16):
                    slj = pl.ds(j * 16, 16)
                    rows[b, k, slj] = rows[b, k, slj] * pk
            pltpu.async_copy(rows.at[b], acc.at[sidx.at[b]], sem_r[b],
                             add=True)
            pltpu.async_copy(pbuf.at[b], sacc.at[sidx.at[b]], sem_p[b],
                             add=True)

            @pl.when(w + 2 < nwin)
            def _():
                issue_idx(w + 2, b)
        return carry

    lax.fori_loop(0, nwin // 2, pair, 0)
    wait_scatters(0)
    wait_scatters(1)


def _sc1_body(h2cat, s1t, d1t, src_a, dst_a, shifts, zrow, zs,
              out, out_s,
              isrc, idst, idxg, sidx, tsrc, tdst, pbuf, rows, shiftb,
              s1sp, d1sp, acc, sacc,
              sem_i0, sem_i1, sem_t0, sem_t1, sem_g0, sem_g1,
              sem_r0, sem_r1, sem_p0, sem_p1):
    c = lax.axis_index("c")
    s = lax.axis_index("s")
    zbase = s * ZR
    lanes0 = lax.iota(jnp.int32, 16) * 0
    pltpu.sync_copy(shifts, shiftb)
    sh_all = shiftb[...]
    bufs = (isrc, idst, idxg, sidx, tsrc, tdst, pbuf, rows)
    sems = ((sem_i0, sem_i1), (sem_t0, sem_t1), (sem_g0, sem_g1),
            (sem_r0, sem_r1), (sem_p0, sem_p1))

    for hh in range(2):
        h = c * 2 + hh
        shift_v = jnp.where(c == 0, sh_all[hh], sh_all[2 + hh])
        pltpu.sync_copy(s1t.at[pl.ds(h * NPAD + zbase, ZR)],
                        s1sp.at[pl.ds(zbase, ZR)])
        pltpu.sync_copy(d1t.at[pl.ds(h * NPAD + zbase, ZR)],
                        d1sp.at[pl.ds(zbase, ZR)])
        pltpu.sync_copy(zrow, acc.at[pl.ds(zbase, ZR)])
        pltpu.sync_copy(zs, sacc.at[pl.ds(zbase, ZR)])
        plsc.subcore_barrier()

        _edge_phase(h2cat, src_a, dst_a, s * W1C, W1C, h * N, shift_v,
                    s1sp, d1sp, bufs, acc, sacc, sems, lanes0)

        plsc.subcore_barrier()
        pltpu.sync_copy(acc.at[pl.ds(zbase, ZR)],
                        out.at[pl.ds(h * NPAD + zbase, ZR)])
        pltpu.sync_copy(sacc.at[pl.ds(zbase, ZR)],
                        out_s.at[pl.ds(h * NPAD + zbase, ZR)])
        plsc.subcore_barrier()


def _sc2_body(g_hbm, s2t, d2t, src_a, dst_a, shifts, zrow, zs,
              out, out_s,
              isrc, idst, idxg, sidx, tsrc, tdst, pbuf, rows, shiftb,
              s2sp, d2sp, acc, sacc,
              sem_i0, sem_i1, sem_t0, sem_t1, sem_g0, sem_g1,
              sem_r0, sem_r1, sem_p0, sem_p1):
    c = lax.axis_index("c")
    s = lax.axis_index("s")
    wid = c * NS + s
    zbase = s * ZR
    lanes0 = lax.iota(jnp.int32, 16) * 0
    pltpu.sync_copy(shifts, shiftb)
    shift_v = shiftb[...][0]
    bufs = (isrc, idst, idxg, sidx, tsrc, tdst, pbuf, rows)
    sems = ((sem_i0, sem_i1), (sem_t0, sem_t1), (sem_g0, sem_g1),
            (sem_r0, sem_r1), (sem_p0, sem_p1))

    pltpu.sync_copy(s2t.at[pl.ds(zbase, ZR)], s2sp.at[pl.ds(zbase, ZR)])
    pltpu.sync_copy(d2t.at[pl.ds(zbase, ZR)], d2sp.at[pl.ds(zbase, ZR)])
    pltpu.sync_copy(zrow, acc.at[pl.ds(zbase, ZR)])
    pltpu.sync_copy(zs, sacc.at[pl.ds(zbase, ZR)])
    plsc.subcore_barrier()

    _edge_phase(g_hbm, src_a, dst_a, wid * W2C, W2C, None, shift_v,
                s2sp, d2sp, bufs, acc, sacc, sems, lanes0)

    plsc.subcore_barrier()
    pltpu.sync_copy(acc.at[pl.ds(zbase, ZR)],
                    out.at[pl.ds(c * NPAD + zbase, ZR)])
    pltpu.sync_copy(sacc.at[pl.ds(zbase, ZR)],
                    out_s.at[pl.ds(c * NPAD + zbase, ZR)])


_MESH = plsc.VectorSubcoreMesh(
    core_axis_name="c", subcore_axis_name="s", num_cores=NC, num_subcores=NS)

_SC_PARAMS = pltpu.CompilerParams(
    needs_layout_passes=False, use_tc_tiling_on_sc=False)

_SC_SEMS = [pltpu.SemaphoreType.DMA] * 10

_SC_SCRATCH = [
    pltpu.VMEM((2, K), jnp.int32),
    pltpu.VMEM((2, K), jnp.int32),
    pltpu.VMEM((2, K), jnp.int32),
    pltpu.VMEM((2, K), jnp.int32),
    pltpu.VMEM((2, K), _f32),
    pltpu.VMEM((2, K), _f32),
    pltpu.VMEM((2, K), _f32),
    pltpu.VMEM((2, K, HID), _f32),
    pltpu.VMEM((16,), _f32),
    pltpu.VMEM_SHARED((NPAD,), _f32),
    pltpu.VMEM_SHARED((NPAD,), _f32),
    pltpu.VMEM_SHARED((NPAD, HID), _f32),
    pltpu.VMEM_SHARED((NPAD,), _f32),
] + _SC_SEMS


def _sc1_call(h2flat, s1t, d1t, srcp, dstp, shifts, zrow, zs):
    return pl.kernel(
        _sc1_body,
        out_type=[
            jax.ShapeDtypeStruct((HEADS * NPAD, HID), _f32),
            jax.ShapeDtypeStruct((HEADS * NPAD,), _f32),
        ],
        mesh=_MESH,
        scratch_types=_SC_SCRATCH,
        compiler_params=_SC_PARAMS,
    )(h2flat, s1t, d1t, srcp, dstp, shifts, zrow, zs)


def _sc2_call(g, s2t, d2t, srcp, dstp, shifts, zrow, zs):
    return pl.kernel(
        _sc2_body,
        out_type=[
            jax.ShapeDtypeStruct((NC * NPAD, OUT_DIM), _f32),
            jax.ShapeDtypeStruct((NC * NPAD,), _f32),
        ],
        mesh=_MESH,
        scratch_types=_SC_SCRATCH,
        compiler_params=_SC_PARAMS,
    )(g, s2t, d2t, srcp, dstp, shifts, zrow, zs)


@jax.jit
def kernel(x, edge_index, W1, a_src1, a_dst1, W2, a_src2, a_dst2):
    loop = jnp.arange(N, dtype=jnp.int32)
    ei = edge_index.astype(jnp.int32)
    src = jnp.concatenate([ei[0], loop])
    dst = jnp.concatenate([ei[1], loop])
    pad = EP - src.shape[0]
    srcp = jnp.concatenate([src, jnp.zeros((pad,), jnp.int32)]).reshape(-1, K)
    dstp = jnp.concatenate([dst, jnp.full((pad,), N, jnp.int32)]).reshape(-1, K)
    zrow = jnp.zeros((ZR, HID), _f32)
    zs = jnp.zeros((ZR,), _f32)

    h2cat, s1, d1, mx1 = _t1_call(x, W1, a_src1, a_dst1)
    h2flat = h2cat.reshape(HEADS * N, HID)
    s1t = jnp.concatenate([s1.T, jnp.zeros((HEADS, NPAD - N), _f32)], 1).reshape(-1)
    d1t = jnp.concatenate([d1.T, jnp.full((HEADS, NPAD - N), -1e30, _f32)], 1).reshape(-1)
    m1 = mx1[0, :4] + mx1[0, 4:]
    shift1 = jnp.maximum(m1, 0.2 * m1)
    shifts1 = jnp.concatenate([shift1, jnp.zeros((12,), _f32)])

    acc1, s1acc = _sc1_call(h2flat, s1t, d1t, srcp, dstp, shifts1, zrow, zs)

    g, sd, mx2 = _t2_call(acc1.reshape(HEADS, NPAD, HID),
                          s1acc.reshape(HEADS, NPAD // 16, 16),
                          W2, a_src2, a_dst2)
    s2t = jnp.concatenate([sd[:, 0], jnp.zeros((NPAD - N,), _f32)])
    d2t = jnp.concatenate([sd[:, 1], jnp.full((NPAD - N,), -1e30, _f32)])
    m2 = mx2[0, 0] + mx2[0, 1]
    shift2 = jnp.maximum(m2, 0.2 * m2)
    shifts2 = jnp.full((16,), shift2, _f32)

    acc2, s2acc = _sc2_call(g, s2t, d2t, srcp, dstp, shifts2, zrow, zs)

    return _t3_call(acc2.reshape(NC, NPAD, OUT_DIM),
                    s2acc.reshape(NC, NPAD // 16, 16))


# speedup vs baseline: 44.9468x; 1.4081x over previous
import jax
import jax.numpy as jnp
from jax import lax
from jax.experimental import pallas as pl
from jax.experimental.pallas import tpu as pltpu
from jax.experimental.pallas import tpu_sc as plsc

N = 10000
IN_DIM = 128
HID = 128
HEADS = 4
OUT_DIM = 128
HH = 64

NC = 2
NS = 16
K = 128

NPAD = 10112
ZR = NPAD // NS
NR = N // NS
EP = 335872
W1C = EP // (NS * K)
W2C = EP // (NS * NC * K)

R_BLK = 400
GRID = N // R_BLK

_f32 = jnp.float32


def _t1_body(x_ref, w1_ref, asrc_ref, adst_ref, h2_ref, s1_ref, d1_ref, mx_ref):
    xb = x_ref[...]
    h2 = lax.dot_general(xb, w1_ref[...], (((1,), (1,)), ((), ())),
                         preferred_element_type=_f32)
    h2r = h2.reshape(R_BLK, HEADS, HID)
    s1 = jnp.sum(h2r * asrc_ref[...][None], axis=-1)
    d1 = jnp.sum(h2r * adst_ref[...][None], axis=-1)
    h2_ref[...] = jnp.concatenate(
        [h2[None, :, i * HH:(i + 1) * HH] for i in range(HEADS * 2)], axis=0)
    s1_ref[...] = s1
    d1_ref[...] = d1
    bm = jnp.concatenate([jnp.max(s1, axis=0), jnp.max(d1, axis=0)]).reshape(1, 8)

    @pl.when(pl.program_id(0) == 0)
    def _():
        mx_ref[...] = bm

    @pl.when(pl.program_id(0) > 0)
    def _():
        mx_ref[...] = jnp.maximum(mx_ref[...], bm)


def _t1_call(x, w1, asrc, adst):
    return pl.pallas_call(
        _t1_body,
        grid=(GRID,),
        in_specs=[
            pl.BlockSpec((R_BLK, IN_DIM), lambda i: (i, 0)),
            pl.BlockSpec((HEADS * HID, IN_DIM), lambda i: (0, 0)),
            pl.BlockSpec((HEADS, HID), lambda i: (0, 0)),
            pl.BlockSpec((HEADS, HID), lambda i: (0, 0)),
        ],
        out_specs=[
            pl.BlockSpec((HEADS * 2, R_BLK, HH), lambda i: (0, i, 0)),
            pl.BlockSpec((R_BLK, HEADS), lambda i: (i, 0)),
            pl.BlockSpec((R_BLK, HEADS), lambda i: (i, 0)),
            pl.BlockSpec((1, 8), lambda i: (0, 0)),
        ],
        out_shape=[
            jax.ShapeDtypeStruct((HEADS * 2, N, HH), _f32),
            jax.ShapeDtypeStruct((N, HEADS), _f32),
            jax.ShapeDtypeStruct((N, HEADS), _f32),
            jax.ShapeDtypeStruct((1, 8), _f32),
        ],
    )(x, w1, asrc, adst)


def _t2_body(acc_ref, s_ref, w2_ref, a2s_ref, a2d_ref, g_ref, sd_ref, mx_ref):
    a = acc_ref[...]
    i0 = pl.program_id(0)
    sv = s_ref[:, pl.ds(i0 * (R_BLK // 16), R_BLK // 16), :]
    r = 1.0 / (sv + 1e-16)
    parts = []
    for i in range(HEADS * 2):
        ai = a[i].reshape(R_BLK // 16, 16, HH)
        hi = jnp.maximum(ai * r[i // 2][:, :, None], 0.0)
        parts.append(hi.reshape(R_BLK, HH))
    h1 = jnp.concatenate(parts, axis=-1)
    g = lax.dot_general(h1, w2_ref[...], (((1,), (1,)), ((), ())),
                        preferred_element_type=_f32)
    g_ref[...] = jnp.concatenate([g[None, :, :HH], g[None, :, HH:]], axis=0)
    s2 = jnp.sum(g * a2s_ref[...], axis=-1)
    d2 = jnp.sum(g * a2d_ref[...], axis=-1)
    sd = jnp.concatenate(
        [s2[:, None], d2[:, None], jnp.zeros((R_BLK, 6), _f32)], axis=1)
    sd_ref[...] = sd
    bm = jnp.concatenate(
        [jnp.max(s2)[None], jnp.max(d2)[None], jnp.zeros((6,), _f32)]).reshape(1, 8)

    @pl.when(pl.program_id(0) == 0)
    def _():
        mx_ref[...] = bm

    @pl.when(pl.program_id(0) > 0)
    def _():
        mx_ref[...] = jnp.maximum(mx_ref[...], bm)


def _t2_call(acc, s, w2, a2s, a2d):
    return pl.pallas_call(
        _t2_body,
        grid=(GRID,),
        in_specs=[
            pl.BlockSpec((HEADS * 2, R_BLK, HH), lambda i: (0, i, 0)),
            pl.BlockSpec((HEADS, NPAD // 16, 16), lambda i: (0, 0, 0)),
            pl.BlockSpec((OUT_DIM, HEADS * HID), lambda i: (0, 0)),
            pl.BlockSpec((1, OUT_DIM), lambda i: (0, 0)),
            pl.BlockSpec((1, OUT_DIM), lambda i: (0, 0)),
        ],
        out_specs=[
            pl.BlockSpec((2, R_BLK, HH), lambda i: (0, i, 0)),
            pl.BlockSpec((R_BLK, 8), lambda i: (i, 0)),
            pl.BlockSpec((1, 8), lambda i: (0, 0)),
        ],
        out_shape=[
            jax.ShapeDtypeStruct((2, N, HH), _f32),
            jax.ShapeDtypeStruct((N, 8), _f32),
            jax.ShapeDtypeStruct((1, 8), _f32),
        ],
    )(acc, s, w2, a2s, a2d)


def _t3_body(p_ref, s_ref, out_ref):
    p = p_ref[...]
    i0 = pl.program_id(0)
    sv = s_ref[:, pl.ds(i0 * (R_BLK // 16), R_BLK // 16), :]
    r = 1.0 / (sv[0] + sv[1] + 1e-16)
    o = []
    for half in range(2):
        m = (p[half] + p[2 + half]).reshape(R_BLK // 16, 16, HH) * r[:, :, None]
        o.append(m.reshape(R_BLK, HH))
    out_ref[...] = jnp.concatenate(o, axis=-1)


def _t3_call(p, s):
    return pl.pallas_call(
        _t3_body,
        grid=(GRID,),
        in_specs=[
            pl.BlockSpec((NC * 2, R_BLK, HH), lambda i: (0, i, 0)),
            pl.BlockSpec((NC, NPAD // 16, 16), lambda i: (0, 0, 0)),
        ],
        out_specs=pl.BlockSpec((R_BLK, OUT_DIM), lambda i: (i, 0)),
        out_shape=jax.ShapeDtypeStruct((N, OUT_DIM), _f32),
    )(p, s)


def _edge_phase(h2sp, src_a, dst_a, rowbase, nwin, shift_v, do_p,
                s_sp, d_sp, bufs, acc, sacc, sems, lanes0):
    isrc, idst, sidx, tsrc, tdst, pbuf, pcache, rows = bufs
    sem_i, sem_t, sem_g, sem_r, sem_p = sems

    def issue_idx(w, b):
        pltpu.async_copy(src_a.at[pl.ds(rowbase + w, 1)], isrc.at[pl.ds(b, 1)],
                         sem_i[b])
        pltpu.async_copy(dst_a.at[pl.ds(rowbase + w, 1)], idst.at[pl.ds(b, 1)],
                         sem_i[b])

    def wait_idx(b):
        pltpu.make_async_copy(src_a.at[pl.ds(0, 1)], isrc.at[pl.ds(b, 1)],
                              sem_i[b]).wait()
        pltpu.make_async_copy(dst_a.at[pl.ds(0, 1)], idst.at[pl.ds(b, 1)],
                              sem_i[b]).wait()

    def issue_gathers(b):
        pltpu.async_copy(h2sp.at[isrc.at[b]], rows.at[b], sem_g[b])
        if do_p:
            pltpu.async_copy(s_sp.at[isrc.at[b]], tsrc.at[b], sem_t[b])
            pltpu.async_copy(d_sp.at[idst.at[b]], tdst.at[b], sem_t[b])

    def wait_gathers(b):
        pltpu.make_async_copy(h2sp.at[isrc.at[b]], rows.at[b], sem_g[b]).wait()
        if do_p:
            pltpu.make_async_copy(s_sp.at[isrc.at[b]], tsrc.at[b],
                                  sem_t[b]).wait()
            pltpu.make_async_copy(d_sp.at[idst.at[b]], tdst.at[b],
                                  sem_t[b]).wait()

    def wait_scatters(b):
        pltpu.make_async_copy(rows.at[b], acc.at[sidx.at[b]], sem_r[b]).wait()
        if do_p:
            pltpu.make_async_copy(pbuf.at[b], sacc.at[sidx.at[b]],
                                  sem_p[b]).wait()

    issue_idx(0, 0)
    issue_idx(1, 1)
    wait_idx(0)
    issue_gathers(0)

    def pair(wp, carry):
        for b in range(2):
            w = wp * 2 + b
            nb = 1 - b

            @pl.when(w + 1 < nwin)
            def _():
                wait_idx(nb)

                @pl.when(w >= 1)
                def _():
                    wait_scatters(nb)

                issue_gathers(nb)

            wait_gathers(b)
            for i in range(K // 16):
                sl = pl.ds(i * 16, 16)
                sidx[b, sl] = idst[b, sl]
                if do_p:
                    t = tsrc[b, sl] + tdst[b, sl]
                    t = jnp.maximum(t, 0.2 * t) - shift_v
                    v = jnp.exp(t)
                    pbuf[b, sl] = v
                    pcache[pl.ds(w * K + i * 16, 16)] = v

            if do_p:
                @plsc.parallel_loop(0, K, unroll=8)
                def _(k):
                    pk = plsc.load_gather(pbuf.at[b], [lanes0 + k])
                    for j in range(HH // 16):
                        slj = pl.ds(j * 16, 16)
                        rows[b, k, slj] = rows[b, k, slj] * pk
            else:
                @plsc.parallel_loop(0, K, unroll=8)
                def _(k):
                    pk = plsc.load_gather(pcache, [lanes0 + (w * K + k)])
                    for j in range(HH // 16):
                        slj = pl.ds(j * 16, 16)
                        rows[b, k, slj] = rows[b, k, slj] * pk

            pltpu.async_copy(rows.at[b], acc.at[sidx.at[b]], sem_r[b],
                             add=True)
            if do_p:
                pltpu.async_copy(pbuf.at[b], sacc.at[sidx.at[b]], sem_p[b],
                                 add=True)

            @pl.when(w + 2 < nwin)
            def _():
                issue_idx(w + 2, b)
        return carry

    lax.fori_loop(0, nwin // 2, pair, 0)
    wait_scatters(0)
    wait_scatters(1)


def _sc1_body(h2cat, s1t, d1t, src_a, dst_a, shifts, zrow, zs,
              out, out_s,
              isrc, idst, sidx, tsrc, tdst, pbuf, pcache, rows, shiftb,
              h2sp, s_sp, d_sp, acc, sacc,
              sem_i0, sem_i1, sem_t0, sem_t1, sem_g0, sem_g1,
              sem_r0, sem_r1, sem_p0, sem_p1):
    c = lax.axis_index("c")
    s = lax.axis_index("s")
    zbase = s * ZR
    nbase = s * NR
    lanes0 = lax.iota(jnp.int32, 16) * 0
    pltpu.sync_copy(shifts, shiftb)
    sh_all = shiftb[...]
    bufs = (isrc, idst, sidx, tsrc, tdst, pbuf, pcache, rows)
    sems = ((sem_i0, sem_i1), (sem_t0, sem_t1), (sem_g0, sem_g1),
            (sem_r0, sem_r1), (sem_p0, sem_p1))

    for hh in range(2):
        h = c * 2 + hh
        shift_v = jnp.where(c == 0, sh_all[hh], sh_all[2 + hh])
        pltpu.sync_copy(s1t.at[pl.ds(h * NPAD + zbase, ZR)],
                        s_sp.at[pl.ds(zbase, ZR)])
        pltpu.sync_copy(d1t.at[pl.ds(h * NPAD + zbase, ZR)],
                        d_sp.at[pl.ds(zbase, ZR)])
        pltpu.sync_copy(zs, sacc.at[pl.ds(zbase, ZR)])
        for half in range(2):
            plane = h * 2 + half
            pltpu.sync_copy(h2cat.at[pl.ds(plane * N + nbase, NR)],
                            h2sp.at[pl.ds(nbase, NR)])
            pltpu.sync_copy(zrow, acc.at[pl.ds(zbase, ZR)])
            plsc.subcore_barrier()

            _edge_phase(h2sp, src_a, dst_a, s * W1C, W1C, shift_v, half == 0,
                        s_sp, d_sp, bufs, acc, sacc, sems, lanes0)

            plsc.subcore_barrier()
            pltpu.sync_copy(acc.at[pl.ds(zbase, ZR)],
                            out.at[pl.ds(plane * NPAD + zbase, ZR)])
            if half == 0:
                pltpu.sync_copy(sacc.at[pl.ds(zbase, ZR)],
                                out_s.at[pl.ds(h * NPAD + zbase, ZR)])
            plsc.subcore_barrier()


def _sc2_body(g2, s2t, d2t, src_a, dst_a, shifts, zrow, zs,
              out, out_s,
              isrc, idst, sidx, tsrc, tdst, pbuf, pcache, rows, shiftb,
              h2sp, s_sp, d_sp, acc, sacc,
              sem_i0, sem_i1, sem_t0, sem_t1, sem_g0, sem_g1,
              sem_r0, sem_r1, sem_p0, sem_p1):
    c = lax.axis_index("c")
    s = lax.axis_index("s")
    wid = c * NS + s
    zbase = s * ZR
    nbase = s * NR
    lanes0 = lax.iota(jnp.int32, 16) * 0
    pltpu.sync_copy(shifts, shiftb)
    shift_v = shiftb[...][0]
    bufs = (isrc, idst, sidx, tsrc, tdst, pbuf, pcache, rows)
    sems = ((sem_i0, sem_i1), (sem_t0, sem_t1), (sem_g0, sem_g1),
            (sem_r0, sem_r1), (sem_p0, sem_p1))

    pltpu.sync_copy(s2t.at[pl.ds(zbase, ZR)], s_sp.at[pl.ds(zbase, ZR)])
    pltpu.sync_copy(d2t.at[pl.ds(zbase, ZR)], d_sp.at[pl.ds(zbase, ZR)])
    pltpu.sync_copy(zs, sacc.at[pl.ds(zbase, ZR)])
    for half in range(2):
        pltpu.sync_copy(g2.at[pl.ds(half * N + nbase, NR)],
                        h2sp.at[pl.ds(nbase, NR)])
        pltpu.sync_copy(zrow, acc.at[pl.ds(zbase, ZR)])
        plsc.subcore_barrier()

        _edge_phase(h2sp, src_a, dst_a, wid * W2C, W2C, shift_v, half == 0,
                    s_sp, d_sp, bufs, acc, sacc, sems, lanes0)

        plsc.subcore_barrier()
        pltpu.sync_copy(acc.at[pl.ds(zbase, ZR)],
                        out.at[pl.ds((c * 2 + half) * NPAD + zbase, ZR)])
        if half == 0:
            pltpu.sync_copy(sacc.at[pl.ds(zbase, ZR)],
                            out_s.at[pl.ds(c * NPAD + zbase, ZR)])
        plsc.subcore_barrier()


_MESH = plsc.VectorSubcoreMesh(
    core_axis_name="c", subcore_axis_name="s", num_cores=NC, num_subcores=NS)

_SC_PARAMS = pltpu.CompilerParams(
    needs_layout_passes=False, use_tc_tiling_on_sc=False)

_SC_SCRATCH = [
    pltpu.VMEM((2, K), jnp.int32),
    pltpu.VMEM((2, K), jnp.int32),
    pltpu.VMEM((2, K), jnp.int32),
    pltpu.VMEM((2, K), _f32),
    pltpu.VMEM((2, K), _f32),
    pltpu.VMEM((2, K), _f32),
    pltpu.VMEM((W1C * K,), _f32),
    pltpu.VMEM((2, K, HH), _f32),
    pltpu.VMEM((16,), _f32),
    pltpu.VMEM_SHARED((N, HH), _f32),
    pltpu.VMEM_SHARED((NPAD,), _f32),
    pltpu.VMEM_SHARED((NPAD,), _f32),
    pltpu.VMEM_SHARED((NPAD, HH), _f32),
    pltpu.VMEM_SHARED((NPAD,), _f32),
] + [pltpu.SemaphoreType.DMA] * 10


def _sc1_call(h2flat, s1t, d1t, srcp, dstp, shifts, zrow, zs):
    return pl.kernel(
        _sc1_body,
        out_type=[
            jax.ShapeDtypeStruct((HEADS * 2 * NPAD, HH), _f32),
            jax.ShapeDtypeStruct((HEADS * NPAD,), _f32),
        ],
        mesh=_MESH,
        scratch_types=_SC_SCRATCH,
        compiler_params=_SC_PARAMS,
    )(h2flat, s1t, d1t, srcp, dstp, shifts, zrow, zs)


def _sc2_call(g, s2t, d2t, srcp, dstp, shifts, zrow, zs):
    return pl.kernel(
        _sc2_body,
        out_type=[
            jax.ShapeDtypeStruct((NC * 2 * NPAD, HH), _f32),
            jax.ShapeDtypeStruct((NC * NPAD,), _f32),
        ],
        mesh=_MESH,
        scratch_types=_SC_SCRATCH,
        compiler_params=_SC_PARAMS,
    )(g, s2t, d2t, srcp, dstp, shifts, zrow, zs)


@jax.jit
def kernel(x, edge_index, W1, a_src1, a_dst1, W2, a_src2, a_dst2):
    loop = jnp.arange(N, dtype=jnp.int32)
    ei = edge_index.astype(jnp.int32)
    src = jnp.concatenate([ei[0], loop])
    dst = jnp.concatenate([ei[1], loop])
    pad = EP - src.shape[0]
    pad_i = jnp.arange(pad, dtype=jnp.int32)
    srcp = jnp.concatenate([src, pad_i % N]).reshape(-1, K)
    dstp = jnp.concatenate([dst, N + pad_i % (NPAD - N)]).reshape(-1, K)
    zrow = jnp.zeros((ZR, HH), _f32)
    zs = jnp.zeros((ZR,), _f32)

    h2cat, s1, d1, mx1 = _t1_call(x, W1, a_src1, a_dst1)
    h2flat = h2cat.reshape(HEADS * 2 * N, HH)
    s1t = jnp.concatenate([s1.T, jnp.zeros((HEADS, NPAD - N), _f32)], 1).reshape(-1)
    d1t = jnp.concatenate([d1.T, jnp.full((HEADS, NPAD - N), -1e30, _f32)], 1).reshape(-1)
    m1 = mx1[0, :4] + mx1[0, 4:]
    shift1 = jnp.maximum(m1, 0.2 * m1)
    shifts1 = jnp.concatenate([shift1, jnp.zeros((12,), _f32)])

    acc1, s1acc = _sc1_call(h2flat, s1t, d1t, srcp, dstp, shifts1, zrow, zs)

    g2, sd, mx2 = _t2_call(acc1.reshape(HEADS * 2, NPAD, HH),
                           s1acc.reshape(HEADS, NPAD // 16, 16),
                           W2, a_src2, a_dst2)
    s2t = jnp.concatenate([sd[:, 0], jnp.zeros((NPAD - N,), _f32)])
    d2t = jnp.concatenate([sd[:, 1], jnp.full((NPAD - N,), -1e30, _f32)])
    m2 = mx2[0, 0] + mx2[0, 1]
    shift2 = jnp.maximum(m2, 0.2 * m2)
    shifts2 = jnp.full((16,), shift2, _f32)

    acc2, s2acc = _sc2_call(g2.reshape(2 * N, HH), s2t, d2t, srcp, dstp,
                            shifts2, zrow, zs)

    return _t3_call(acc2.reshape(NC * 2, NPAD, HH),
                    s2acc.reshape(NC, NPAD // 16, 16))
